# Initial kernel scaffold; baseline (speedup 1.0000x reference)
import jax
import jax.numpy as jnp
from jax.experimental import pallas as pl


def kernel(sequences, a, b):
    raise NotImplementedError("write your pallas kernel here")


# speedup vs baseline: 3.3942x; 3.3942x over previous
import functools

import jax
import jax.numpy as jnp
from jax import lax
from jax.experimental import pallas as pl
from jax.experimental.pallas import tpu as pltpu
from jax.experimental.pallas import tpu_sc as plsc

_P = (1 << 31) - 1
_B = 16
_N = 2048
_L = 16
_CHUNKS = _N // _L


def _mod_p(v):
    v = (v >> 31) + (v & _P)
    return jnp.where(v >= _P, v - _P, v)


def _hash_body(seq_hbm, a_hbm, b_hbm, out_hbm, seq_v, a_v, b_v, out_v):
    wid = lax.axis_index("s") * 2 + lax.axis_index("c")

    @pl.when(wid < _B)
    def _():
        pltpu.sync_copy(seq_hbm.at[wid], seq_v)
        pltpu.sync_copy(a_hbm, a_v)
        pltpu.sync_copy(b_hbm, b_v)
        b_vec = b_v[...].astype(jnp.uint32)

        def body(k, carry):
            c0, c1, nz = carry
            base = k * jnp.int32(_L)
            xi = seq_v[pl.ds(base, _L)]
            ai = a_v[pl.ds(base, _L)]
            x = xi.astype(jnp.uint32)
            a = ai.astype(jnp.uint32)
            x0 = x & 0xFFFF
            x1 = x >> 16
            a0 = a & 0xFFFF
            a1 = a >> 16
            p00 = x0 * a0
            mid = x1 * a0 + x0 * a1
            p11 = x1 * a1
            t_mid = (mid >> 15) + ((mid & 0x7FFF) << 16)
            t_mid = jnp.where(t_mid >= _P, t_mid - _P, t_mid)
            t00 = (p00 >> 31) + (p00 & _P)
            t00 = jnp.where(t00 >= _P, t00 - _P, t00)
            term = _mod_p(2 * p11 + t_mid + t00)
            d0 = term & 0xFFFF
            d1 = term >> 16
            s0 = plsc.cumsum(d0) + c0
            s1 = plsc.cumsum(d1) + c1
            u = (s1 >> 15) + ((s1 & 0x7FFF) << 16) + s0
            u = _mod_p(u)
            w = u + b_vec
            w = jnp.where(w >= _P, w - _P, w)
            out = (w & 0xFFFF) + 1
            out_v[pl.ds(base, _L)] = out.astype(jnp.int32)
            nz = nz + jnp.sum((xi != 0).astype(jnp.int32), dtype=jnp.int32)
            return (
                c0 + jnp.sum(d0, dtype=jnp.uint32),
                c1 + jnp.sum(d1, dtype=jnp.uint32),
                nz,
            )

        _, _, nz = lax.fori_loop(
            jnp.int32(0),
            jnp.int32(_CHUNKS),
            body,
            (jnp.uint32(0), jnp.uint32(0), jnp.int32(0)),
        )

        last_idx = jnp.clip(nz - jnp.int32(1), jnp.int32(0), jnp.int32(_N - 1))
        idxs = jnp.zeros((_L,), jnp.int32) + last_idx
        last_val = plsc.load_gather(out_v, [idxs])
        k0 = lax.shift_right_logical(nz, jnp.int32(4))

        def pad_body(k, carry):
            base = k * jnp.int32(_L)
            pos = base + lax.iota(jnp.int32, _L)
            cur = out_v[pl.ds(base, _L)]
            out_v[pl.ds(base, _L)] = jnp.where(pos >= nz, last_val, cur)
            return carry

        lax.fori_loop(k0, jnp.int32(_CHUNKS), pad_body, jnp.int32(0))
        pltpu.sync_copy(out_v, out_hbm.at[wid])


_hash_kernel = functools.partial(
    pl.kernel,
    out_type=jax.ShapeDtypeStruct((_B, _N), jnp.int32),
    mesh=plsc.VectorSubcoreMesh(
        core_axis_name="c", subcore_axis_name="s", num_cores=2, num_subcores=16
    ),
    scratch_types=[
        pltpu.VMEM((_N,), jnp.int32),
        pltpu.VMEM((_N,), jnp.int32),
        pltpu.VMEM((_L,), jnp.int32),
        pltpu.VMEM((_N,), jnp.int32),
    ],
    compiler_params=pltpu.CompilerParams(needs_layout_passes=False),
)(_hash_body)


def kernel(sequences, a, b):
    seq32 = sequences.astype(jnp.int32)
    a32 = a.astype(jnp.int32)
    b_vec = jnp.full((_L,), b, jnp.int32)
    out = _hash_kernel(seq32, a32, b_vec)
    return out.astype(jnp.int64)


# speedup vs baseline: 3.4217x; 1.0081x over previous
import functools

import jax
import jax.numpy as jnp
from jax import lax
from jax.experimental import pallas as pl
from jax.experimental.pallas import tpu as pltpu
from jax.experimental.pallas import tpu_sc as plsc

_P = (1 << 31) - 1
_B = 16
_N = 2048
_L = 16
_CHUNKS = _N // _L


def _hash_body(seq_hbm, a_hbm, b_hbm, out_hbm, seq_v, a_v, b_v, out_v):
    wid = lax.axis_index("s") * 2 + lax.axis_index("c")

    @pl.when(wid < _B)
    def _():
        pltpu.sync_copy(seq_hbm.at[wid], seq_v)
        pltpu.sync_copy(a_hbm, a_v)
        pltpu.sync_copy(b_hbm, b_v)
        b_vec = b_v[...].astype(jnp.uint32)
        lane15 = jnp.full((_L,), 15, jnp.int32)
        zero_u = jnp.zeros((_L,), jnp.uint32)

        def take_last(v):
            return v.at[lane15].get(mode="promise_in_bounds")

        def body(k, carry):
            c0, c1, c2, nzv = carry
            base = k * jnp.int32(_L)
            xi = seq_v[pl.ds(base, _L)]
            ai = a_v[pl.ds(base, _L)]
            x = xi.astype(jnp.uint32)
            a = ai.astype(jnp.uint32)
            x0 = x & 0xFFFF
            x1 = x >> 16
            a0 = a & 0xFFFF
            a1 = a >> 16
            p00 = x0 * a0
            mid = x1 * a0 + x0 * a1
            p11 = x1 * a1
            e0 = p00 & 0xFFFF
            e1 = (p00 >> 16) + (mid & 0xFFFF)
            e2 = (mid >> 16) + p11
            s0 = plsc.cumsum(e0) + c0
            s1 = plsc.cumsum(e1) + c1
            s2 = plsc.cumsum(e2) + c2
            u = 2 * s2 + (s1 >> 15) + ((s1 & 0x7FFF) << 16) + s0
            u = (u >> 31) + (u & _P)
            u = jnp.minimum(u, u - _P)
            w = u + b_vec
            w = jnp.minimum(w, w - _P)
            out_v[pl.ds(base, _L)] = ((w & 0xFFFF) + 1).astype(jnp.int32)
            nzv = nzv + jnp.minimum(x, 1)
            return (take_last(s0), take_last(s1), take_last(s2), nzv)

        _, _, _, nzv = lax.fori_loop(
            jnp.int32(0),
            jnp.int32(_CHUNKS),
            body,
            (zero_u, zero_u, zero_u, zero_u),
        )
        nz = jnp.sum(nzv, dtype=jnp.int32)

        last_idx = jnp.clip(nz - jnp.int32(1), jnp.int32(0), jnp.int32(_N - 1))
        idxs = jnp.zeros((_L,), jnp.int32) + last_idx
        last_val = plsc.load_gather(out_v, [idxs])
        k0 = lax.shift_right_logical(nz, jnp.int32(4))

        def pad_body(k, carry):
            base = k * jnp.int32(_L)
            pos = base + lax.iota(jnp.int32, _L)
            cur = out_v[pl.ds(base, _L)]
            out_v[pl.ds(base, _L)] = jnp.where(pos >= nz, last_val, cur)
            return carry

        lax.fori_loop(k0, jnp.int32(_CHUNKS), pad_body, jnp.int32(0))
        pltpu.sync_copy(out_v, out_hbm.at[wid])


_hash_kernel = functools.partial(
    pl.kernel,
    out_type=jax.ShapeDtypeStruct((_B, _N), jnp.int32),
    mesh=plsc.VectorSubcoreMesh(
        core_axis_name="c", subcore_axis_name="s", num_cores=2, num_subcores=16
    ),
    scratch_types=[
        pltpu.VMEM((_N,), jnp.int32),
        pltpu.VMEM((_N,), jnp.int32),
        pltpu.VMEM((_L,), jnp.int32),
        pltpu.VMEM((_N,), jnp.int32),
    ],
    compiler_params=pltpu.CompilerParams(needs_layout_passes=False),
)(_hash_body)


def kernel(sequences, a, b):
    seq32 = sequences.astype(jnp.int32)
    a32 = a.astype(jnp.int32)
    b_vec = jnp.full((_L,), b, jnp.int32)
    out = _hash_kernel(seq32, a32, b_vec)
    return out.astype(jnp.int64)


# speedup vs baseline: 3.6061x; 1.0539x over previous
import functools

import jax
import jax.numpy as jnp
from jax import lax
from jax.experimental import pallas as pl
from jax.experimental.pallas import tpu as pltpu
from jax.experimental.pallas import tpu_sc as plsc

_P = (1 << 31) - 1
_B = 16
_N = 2048
_L = 16
_CHUNKS = _N // _L


def _hash_body(seq_hbm, a_hbm, b_hbm, out_hbm, seq_v, a_v, b_v, out_v):
    wid = lax.axis_index("s")

    pltpu.sync_copy(seq_hbm.at[wid], seq_v)
    pltpu.sync_copy(a_hbm, a_v)
    pltpu.sync_copy(b_hbm, b_v)
    b_vec = b_v[...].astype(jnp.uint32)
    lane15 = jnp.full((_L,), 15, jnp.int32)
    zero_u = jnp.zeros((_L,), jnp.uint32)

    def take_last(v):
        return v.at[lane15].get(mode="promise_in_bounds")

    def body(k, carry):
        c0, c1, c2, nzv = carry
        base = k * jnp.int32(_L)
        xi = seq_v[pl.ds(base, _L)]
        ai = a_v[pl.ds(base, _L)]
        x = xi.astype(jnp.uint32)
        a = ai.astype(jnp.uint32)
        x0 = x & 0xFFFF
        x1 = x >> 16
        a0 = a & 0xFFFF
        a1 = a >> 16
        p00 = x0 * a0
        mid = x1 * a0 + x0 * a1
        p11 = x1 * a1
        e0 = p00 & 0xFFFF
        e1 = (p00 >> 16) + (mid & 0xFFFF)
        e2 = (mid >> 16) + p11
        s0 = plsc.cumsum(e0) + c0
        s1 = plsc.cumsum(e1) + c1
        s2 = plsc.cumsum(e2) + c2
        u = 2 * s2 + (s1 >> 15) + ((s1 & 0x7FFF) << 16) + s0
        u = (u >> 31) + (u & _P)
        u = jnp.minimum(u, u - _P)
        w = u + b_vec
        w = jnp.minimum(w, w - _P)
        out_v[pl.ds(base, _L)] = ((w & 0xFFFF) + 1).astype(jnp.int32)
        nzv = nzv + jnp.minimum(x, 1)
        return (take_last(s0), take_last(s1), take_last(s2), nzv)

    _, _, _, nzv = lax.fori_loop(
        jnp.int32(0),
        jnp.int32(_CHUNKS),
        body,
        (zero_u, zero_u, zero_u, zero_u),
    )
    nz = jnp.sum(nzv, dtype=jnp.int32)

    last_idx = jnp.clip(nz - jnp.int32(1), jnp.int32(0), jnp.int32(_N - 1))
    idxs = jnp.zeros((_L,), jnp.int32) + last_idx
    last_val = plsc.load_gather(out_v, [idxs])
    k0 = lax.shift_right_logical(nz, jnp.int32(4))

    def pad_body(k, carry):
        base = k * jnp.int32(_L)
        pos = base + lax.iota(jnp.int32, _L)
        cur = out_v[pl.ds(base, _L)]
        out_v[pl.ds(base, _L)] = jnp.where(pos >= nz, last_val, cur)
        return carry

    lax.fori_loop(k0, jnp.int32(_CHUNKS), pad_body, jnp.int32(0))
    pltpu.sync_copy(out_v, out_hbm.at[wid])


_hash_kernel = functools.partial(
    pl.kernel,
    out_type=jax.ShapeDtypeStruct((_B, _N), jnp.int32),
    mesh=plsc.VectorSubcoreMesh(
        core_axis_name="c", subcore_axis_name="s", num_cores=1, num_subcores=16
    ),
    scratch_types=[
        pltpu.VMEM((_N,), jnp.int32),
        pltpu.VMEM((_N,), jnp.int32),
        pltpu.VMEM((_L,), jnp.int32),
        pltpu.VMEM((_N,), jnp.int32),
    ],
    compiler_params=pltpu.CompilerParams(needs_layout_passes=False),
)(_hash_body)


def kernel(sequences, a, b):
    seq32 = sequences.astype(jnp.int32)
    a32 = a.astype(jnp.int32)
    b_vec = jnp.full((_L,), b, jnp.int32)
    out = _hash_kernel(seq32, a32, b_vec)
    return out.astype(jnp.int64)


# speedup vs baseline: 3.6073x; 1.0003x over previous
import functools

import jax
import jax.numpy as jnp
from jax import lax
from jax.experimental import pallas as pl
from jax.experimental.pallas import tpu as pltpu
from jax.experimental.pallas import tpu_sc as plsc

_P = (1 << 31) - 1
_B = 16
_N = 2048
_L = 16
_CHUNKS = _N // _L


def _hash_body(seq_hbm, a_hbm, b_hbm, out_hbm, seq_v, a_v, b_v, out_v):
    wid = lax.axis_index("s")

    pltpu.sync_copy(seq_hbm.at[wid], seq_v)
    pltpu.sync_copy(a_hbm, a_v)
    pltpu.sync_copy(b_hbm, b_v)
    b_vec = b_v[...].astype(jnp.uint32)
    lane15 = jnp.full((_L,), 15, jnp.int32)
    zero_u = jnp.zeros((_L,), jnp.uint32)

    def take_last(v):
        return v.at[lane15].get(mode="promise_in_bounds")

    def body(k, carry):
        c0, c1, c2, nzv = carry
        base = k * jnp.int32(_L)
        xi = seq_v[pl.ds(base, _L)]
        ai = a_v[pl.ds(base, _L)]
        x = xi.astype(jnp.uint32)
        a = ai.astype(jnp.uint32)
        x0 = x & 0xFFFF
        x1 = x >> 16
        a0 = a & 0xFFFF
        a1 = a >> 16
        p00 = x0 * a0
        mid = x1 * a0 + x0 * a1
        p11 = x1 * a1
        e0 = p00 & 0xFFFF
        e1 = (p00 >> 16) + (mid & 0xFFFF)
        e2 = (mid >> 16) + p11
        s0 = plsc.cumsum(e0) + c0
        s1 = plsc.cumsum(e1) + c1
        s2 = plsc.cumsum(e2) + c2
        u = 2 * s2 + (s1 >> 15) + ((s1 & 0x7FFF) << 16) + s0
        u = (u >> 31) + (u & _P)
        u = jnp.minimum(u, u - _P)
        w = u + b_vec
        w = jnp.minimum(w, w - _P)
        out_v[pl.ds(base, _L)] = ((w & 0xFFFF) + 1).astype(jnp.int32)
        nzv = nzv + jnp.minimum(x, 1)
        return (take_last(s0), take_last(s1), take_last(s2), nzv)

    _, _, _, nzv = lax.fori_loop(
        jnp.int32(0),
        jnp.int32(_CHUNKS),
        body,
        (zero_u, zero_u, zero_u, zero_u),
    )
    nz = jnp.sum(nzv, dtype=jnp.int32)

    last_idx = jnp.clip(nz - jnp.int32(1), jnp.int32(0), jnp.int32(_N - 1))
    idxs = jnp.zeros((_L,), jnp.int32) + last_idx
    last_val = plsc.load_gather(out_v, [idxs])
    k0 = lax.shift_right_logical(nz, jnp.int32(4))

    def pad_body(k, carry):
        base = k * jnp.int32(_L)
        pos = base + lax.iota(jnp.int32, _L)
        cur = out_v[pl.ds(base, _L)]
        out_v[pl.ds(base, _L)] = jnp.where(pos >= nz, last_val, cur)
        return carry

    lax.fori_loop(k0, jnp.int32(_CHUNKS), pad_body, jnp.int32(0))
    pltpu.sync_copy(out_v, out_hbm.at[wid])


_hash_kernel = functools.partial(
    pl.kernel,
    out_type=jax.ShapeDtypeStruct((_B, _N), jnp.int32),
    mesh=plsc.VectorSubcoreMesh(
        core_axis_name="c", subcore_axis_name="s", num_cores=1, num_subcores=16
    ),
    scratch_types=[
        pltpu.VMEM((_N,), jnp.int32),
        pltpu.VMEM((_N,), jnp.int32),
        pltpu.VMEM((_L,), jnp.int32),
        pltpu.VMEM((_N,), jnp.int32),
    ],
    compiler_params=pltpu.CompilerParams(
        needs_layout_passes=False, skip_device_barrier=True
    ),
)(_hash_body)


def kernel(sequences, a, b):
    seq32 = sequences.astype(jnp.int32)
    a32 = a.astype(jnp.int32)
    b_vec = jnp.full((_L,), b, jnp.int32)
    out = _hash_kernel(seq32, a32, b_vec)
    return out.astype(jnp.int64)
